# baseline (speedup 1.0000x reference)
import functools

import jax
import jax.numpy as jnp
from jax import lax
from jax.experimental import pallas as pl
from jax.experimental.pallas import tpu as pltpu
from jax.experimental.pallas import tpu_sc as plsc

N = 10000
NP = 10240
E = 320000
D = 128
NC = 2
NS = 16
CH = 80
ET = E // NS
TCHUNKS = ET // CH
ET2 = E // (NC * NS)
TCHUNKS2 = ET2 // CH
RPT = NP // NS

_MESH = plsc.VectorSubcoreMesh(
    core_axis_name="c", subcore_axis_name="s", num_cores=NC, num_subcores=NS)


@functools.partial(
    pl.kernel,
    out_type=jax.ShapeDtypeStruct((NC * NP,), jnp.float32),
    mesh=_MESH,
    scratch_types=[
        pltpu.VMEM((ET,), jnp.int32),
        pltpu.VMEM((3, CH), jnp.int32),
        pltpu.VMEM((CH,), jnp.float32),
        pltpu.VMEM_SHARED((NP,), jnp.float32),
        pltpu.SemaphoreType.DMA((3,)),
    ],
)
def _deg_kernel(eidx_hbm, zeros_hbm, ones_hbm, out_hbm,
                idx_v, didx_v, ones_v, deg_sh, sem_s):
    c = lax.axis_index("c")
    s = lax.axis_index("s")

    pltpu.sync_copy(ones_hbm, ones_v)
    base = pl.multiple_of(c * E + s * ET, 8)
    pltpu.sync_copy(eidx_hbm.at[pl.ds(base, ET)], idx_v)
    pltpu.sync_copy(zeros_hbm.at[pl.ds(s * RPT, RPT)], deg_sh.at[pl.ds(s * RPT, RPT)])
    plsc.subcore_barrier()

    def step(j, _):
        p = lax.rem(j, 3)

        @pl.when(j > 1)
        def _():
            q = lax.rem(j + 1, 3)
            pltpu.make_async_copy(ones_v, deg_sh.at[didx_v.at[q]], sem_s.at[q]).wait()

        for i in range(CH // 16):
            didx_v[p, pl.ds(i * 16, 16)] = idx_v[pl.ds(j * CH + i * 16, 16)]
        pltpu.async_copy(ones_v, deg_sh.at[didx_v.at[p]], sem_s.at[p], add=True)
        return 0
    lax.fori_loop(0, TCHUNKS, step, 0)
    for jj in (TCHUNKS - 2, TCHUNKS - 1):
        pltpu.make_async_copy(ones_v, deg_sh.at[didx_v.at[jj % 3]],
                              sem_s.at[jj % 3]).wait()

    plsc.subcore_barrier()
    obase = pl.multiple_of(c * NP + s * RPT, 8)
    pltpu.sync_copy(deg_sh.at[pl.ds(s * RPT, RPT)], out_hbm.at[pl.ds(obase, RPT)])


NROWS = 3
NDIDX = 4
CHS = 128
ESC = E // NC
NCHS = ESC // CHS
BCT = NCHS // NS
XTRA = NCHS % NS
NPA = 10112
RPTA = NPA // NS


@functools.partial(
    pl.kernel,
    out_type=jax.ShapeDtypeStruct((NC * NPA, D), jnp.float32),
    mesh=_MESH,
    scratch_types=[
        pltpu.VMEM((NROWS, CHS), jnp.int32),
        pltpu.VMEM((NDIDX, CHS), jnp.int32),
        pltpu.VMEM((NROWS, CHS, D), jnp.float32),
        pltpu.VMEM_SHARED((NPA, D), jnp.float32),
        pltpu.SemaphoreType.DMA((NDIDX,)),
        pltpu.SemaphoreType.DMA((NROWS,)),
        pltpu.SemaphoreType.DMA((NROWS,)),
    ],
)
def _scatter_kernel(hw_hbm, eidx_hbm, zeros_hbm, out_hbm,
                    sidx_v, didx_v, rows_v, agg_sh, sem_i, sem_g, sem_s):
    c = lax.axis_index("c")
    s = lax.axis_index("s")
    nchunks = BCT + (s < XTRA).astype(jnp.int32)
    cbase = BCT * s + jnp.minimum(s, XTRA)
    tbase = c * ESC + cbase * CHS

    def idx_refs(j, ss, ds):
        sb = pl.ds(pl.multiple_of(tbase + j * CHS, 8), CHS)
        db = pl.ds(pl.multiple_of(E + tbase + j * CHS, 8), CHS)
        return ((eidx_hbm.at[sb], sidx_v.at[ss], sem_i.at[ds]),
                (eidx_hbm.at[db], didx_v.at[ds], sem_i.at[ds]))

    def idx_start(j, ss, ds):
        for tr in idx_refs(j, ss, ds):
            pltpu.async_copy(*tr)

    def idx_wait(j, ss, ds):
        for tr in idx_refs(j, ss, ds):
            pltpu.make_async_copy(*tr).wait()

    def gather_refs(ri):
        return (hw_hbm.at[sidx_v.at[ri]], rows_v.at[ri], sem_g.at[ri])

    def scat_refs(ri, di):
        return (rows_v.at[ri], agg_sh.at[didx_v.at[di]], sem_s.at[ri])

    for jj in range(3):
        idx_start(jj, jj, jj)
    for jj in range(2):
        idx_wait(jj, jj, jj)
        pltpu.async_copy(*gather_refs(jj))

    pltpu.sync_copy(zeros_hbm.at[pl.ds(s * RPTA, RPTA)],
                    agg_sh.at[pl.ds(s * RPTA, RPTA)])
    plsc.subcore_barrier()

    def step(j, _):
        p3 = lax.rem(j, NROWS)
        p4 = lax.rem(j, NDIDX)

        pltpu.make_async_copy(*gather_refs(p3)).wait()

        @pl.when(j > 0)
        def _():
            pltpu.make_async_copy(
                *scat_refs(lax.rem(j + NROWS - 1, NROWS),
                           lax.rem(j + NDIDX - 1, NDIDX))).wait()

        pltpu.async_copy(*scat_refs(p3, p4), add=True)

        @pl.when(j + 3 < nchunks)
        def _():
            idx_start(j + 3, p3, lax.rem(j + 3, NDIDX))

        @pl.when(j + 2 < nchunks)
        def _():
            r3 = lax.rem(j + 2, NROWS)
            idx_wait(j + 2, r3, lax.rem(j + 2, NDIDX))
            pltpu.async_copy(*gather_refs(r3))
        return 0
    lax.fori_loop(0, nchunks, step, 0)
    pltpu.make_async_copy(
        *scat_refs(lax.rem(nchunks - 1, NROWS),
                   lax.rem(nchunks - 1, NDIDX))).wait()

    plsc.subcore_barrier()
    obase = pl.multiple_of(c * NPA + s * RPTA, 8)
    pltpu.sync_copy(agg_sh.at[pl.ds(s * RPTA, RPTA)], out_hbm.at[pl.ds(obase, RPTA)])


def _scale_of(deg_ref):
    return lax.rsqrt(jnp.maximum(deg_ref[...], 1.0))


def _f0_body(x_ref, do_ref, w_ref, out_ref):
    s = lax.rsqrt(jnp.maximum(do_ref[...], 1.0))
    out_ref[...] = lax.dot_general(x_ref[...] * s, w_ref[...],
                                   (((1,), (1,)), ((), ())),
                                   preferred_element_type=jnp.float32)


BR = NPA // 8


def _bn_relu(agg_ref, di_ref, g_ref, b_ref):
    a = agg_ref[...]
    h = a[0:N] + a[NPA:NPA + N]
    h = h * _scale_of(di_ref)
    m = jnp.mean(h, axis=0, keepdims=True)
    d = h - m
    v = jnp.mean(d * d, axis=0, keepdims=True)
    hn = d * lax.rsqrt(v + 1e-5) * g_ref[...] + b_ref[...]
    return jnp.maximum(hn, 0.0)


def _f1_body(agg_ref, do_ref, di_ref, g_ref, b_ref, w_ref, out_ref):
    hr = _bn_relu(agg_ref, di_ref, g_ref, b_ref)
    out_ref[0:N, :] = lax.dot_general(hr * _scale_of(do_ref), w_ref[...],
                                      (((1,), (1,)), ((), ())),
                                      preferred_element_type=jnp.float32)


def _fc_body(agg_ref, di_ref, g_ref, b_ref, wc_ref, bc_ref, out_ref):
    hr = _bn_relu(agg_ref, di_ref, g_ref, b_ref)
    out_ref[...] = lax.dot_general(hr, wc_ref[...],
                                   (((1,), (1,)), ((), ())),
                                   preferred_element_type=jnp.float32) + bc_ref[...]


_f0 = pl.pallas_call(
    _f0_body,
    grid=(NPA // BR,),
    in_specs=[
        pl.BlockSpec((BR, D), lambda g: (g, 0)),
        pl.BlockSpec((BR, 1), lambda g: (g, 0)),
        pl.BlockSpec((D, D), lambda g: (0, 0)),
    ],
    out_specs=pl.BlockSpec((BR, D), lambda g: (g, 0)),
    out_shape=jax.ShapeDtypeStruct((NPA, D), jnp.float32),
)
_f1 = pl.pallas_call(_f1_body, out_shape=jax.ShapeDtypeStruct((NP, D), jnp.float32))
_fc = pl.pallas_call(_fc_body, out_shape=jax.ShapeDtypeStruct((N, 40), jnp.float32))


def kernel(x, edge_index, W0, g0, b0, W1, g1, b1, W2, g2, b2, Wc, bc):
    eidx = edge_index.reshape(2 * E)
    zeros1 = jnp.zeros((NP,), jnp.float32)
    zeros128 = jnp.zeros((NPA, D), jnp.float32)
    g0r, g1r, g2r = g0.reshape(1, D), g1.reshape(1, D), g2.reshape(1, D)
    b0r, b1r, b2r = b0.reshape(1, D), b1.reshape(1, D), b2.reshape(1, D)
    bcr = bc.reshape(1, 40)

    degs = _deg_kernel(eidx, zeros1, jnp.ones((CH,), jnp.float32))
    do = degs[0:N].reshape(N, 1)
    di = degs[NP:NP + N].reshape(N, 1)

    xp = jnp.pad(x, ((0, NPA - N), (0, 0)))
    dop = jnp.pad(degs[0:N], (0, NPA - N)).reshape(NPA, 1)
    hw = _f0(xp, dop, W0)
    agg = _scatter_kernel(hw, eidx, zeros128)
    hw = _f1(agg, do, di, g0r, b0r, W1)
    agg = _scatter_kernel(hw, eidx, zeros128)
    hw = _f1(agg, do, di, g1r, b1r, W2)
    agg = _scatter_kernel(hw, eidx, zeros128)
    return _fc(agg, di, g2r, b2r, Wc, bcr)


# speedup vs baseline: 1.0182x; 1.0182x over previous
import functools

import jax
import jax.numpy as jnp
from jax import lax
from jax.experimental import pallas as pl
from jax.experimental.pallas import tpu as pltpu
from jax.experimental.pallas import tpu_sc as plsc

N = 10000
NP = 10240
E = 320000
D = 128
NC = 2
NS = 16
CH = 80
ET = E // NS
TCHUNKS = ET // CH
ET2 = E // (NC * NS)
TCHUNKS2 = ET2 // CH
RPT = NP // NS

_MESH = plsc.VectorSubcoreMesh(
    core_axis_name="c", subcore_axis_name="s", num_cores=NC, num_subcores=NS)


@functools.partial(
    pl.kernel,
    out_type=jax.ShapeDtypeStruct((NC * NP,), jnp.float32),
    mesh=_MESH,
    scratch_types=[
        pltpu.VMEM((ET,), jnp.int32),
        pltpu.VMEM((3, CH), jnp.int32),
        pltpu.VMEM((CH,), jnp.float32),
        pltpu.VMEM_SHARED((NP,), jnp.float32),
        pltpu.SemaphoreType.DMA((3,)),
    ],
)
def _deg_kernel(eidx_hbm, zeros_hbm, ones_hbm, out_hbm,
                idx_v, didx_v, ones_v, deg_sh, sem_s):
    c = lax.axis_index("c")
    s = lax.axis_index("s")

    pltpu.sync_copy(ones_hbm, ones_v)
    base = pl.multiple_of(c * E + s * ET, 8)
    pltpu.sync_copy(eidx_hbm.at[pl.ds(base, ET)], idx_v)
    pltpu.sync_copy(zeros_hbm.at[pl.ds(s * RPT, RPT)], deg_sh.at[pl.ds(s * RPT, RPT)])
    plsc.subcore_barrier()

    def step(j, _):
        p = lax.rem(j, 3)

        @pl.when(j > 1)
        def _():
            q = lax.rem(j + 1, 3)
            pltpu.make_async_copy(ones_v, deg_sh.at[didx_v.at[q]], sem_s.at[q]).wait()

        for i in range(CH // 16):
            didx_v[p, pl.ds(i * 16, 16)] = idx_v[pl.ds(j * CH + i * 16, 16)]
        pltpu.async_copy(ones_v, deg_sh.at[didx_v.at[p]], sem_s.at[p], add=True)
        return 0
    lax.fori_loop(0, TCHUNKS, step, 0)
    for jj in (TCHUNKS - 2, TCHUNKS - 1):
        pltpu.make_async_copy(ones_v, deg_sh.at[didx_v.at[jj % 3]],
                              sem_s.at[jj % 3]).wait()

    plsc.subcore_barrier()
    obase = pl.multiple_of(c * NP + s * RPT, 8)
    pltpu.sync_copy(deg_sh.at[pl.ds(s * RPT, RPT)], out_hbm.at[pl.ds(obase, RPT)])


NROWS = 3
NDIDX = 4
CHS = 128
ESC = E // NC
NCHS = ESC // CHS
BCT = NCHS // NS
XTRA = NCHS % NS
NPA = 10112
RPTA = NPA // NS


@functools.partial(
    pl.kernel,
    out_type=jax.ShapeDtypeStruct((NC * NPA, D), jnp.float32),
    mesh=_MESH,
    scratch_types=[
        pltpu.VMEM((NROWS, CHS), jnp.int32),
        pltpu.VMEM((NDIDX, CHS), jnp.int32),
        pltpu.VMEM((NROWS, CHS, D), jnp.float32),
        pltpu.VMEM_SHARED((NPA, D), jnp.float32),
        pltpu.SemaphoreType.DMA((NDIDX,)),
        pltpu.SemaphoreType.DMA((NROWS,)),
        pltpu.SemaphoreType.DMA((NROWS,)),
    ],
)
def _scatter_kernel(hw_hbm, eidx_hbm, zeros_hbm, out_hbm,
                    sidx_v, didx_v, rows_v, agg_sh, sem_i, sem_g, sem_s):
    c = lax.axis_index("c")
    s = lax.axis_index("s")
    nchunks = BCT + (s < XTRA).astype(jnp.int32)
    cbase = BCT * s + jnp.minimum(s, XTRA)
    tbase = c * ESC + cbase * CHS

    def idx_refs(j, ss, ds):
        sb = pl.ds(pl.multiple_of(tbase + j * CHS, 8), CHS)
        db = pl.ds(pl.multiple_of(E + tbase + j * CHS, 8), CHS)
        return ((eidx_hbm.at[sb], sidx_v.at[ss], sem_i.at[ds]),
                (eidx_hbm.at[db], didx_v.at[ds], sem_i.at[ds]))

    def idx_start(j, ss, ds):
        for tr in idx_refs(j, ss, ds):
            pltpu.async_copy(*tr)

    def idx_wait(j, ss, ds):
        for tr in idx_refs(j, ss, ds):
            pltpu.make_async_copy(*tr).wait()

    def gather_refs(ri):
        return (hw_hbm.at[sidx_v.at[ri]], rows_v.at[ri], sem_g.at[ri])

    def scat_refs(ri, di):
        return (rows_v.at[ri], agg_sh.at[didx_v.at[di]], sem_s.at[ri])

    for jj in range(3):
        idx_start(jj, jj, jj)
    for jj in range(2):
        idx_wait(jj, jj, jj)
        pltpu.async_copy(*gather_refs(jj))

    pltpu.sync_copy(zeros_hbm.at[pl.ds(s * RPTA, RPTA)],
                    agg_sh.at[pl.ds(s * RPTA, RPTA)])
    plsc.subcore_barrier()

    def step(j, _):
        p3 = lax.rem(j, NROWS)
        p4 = lax.rem(j, NDIDX)

        pltpu.make_async_copy(*gather_refs(p3)).wait()

        @pl.when(j > 0)
        def _():
            pltpu.make_async_copy(
                *scat_refs(lax.rem(j + NROWS - 1, NROWS),
                           lax.rem(j + NDIDX - 1, NDIDX))).wait()

        pltpu.async_copy(*scat_refs(p3, p4), add=True)

        @pl.when(j + 3 < nchunks)
        def _():
            idx_start(j + 3, p3, lax.rem(j + 3, NDIDX))

        @pl.when(j + 2 < nchunks)
        def _():
            r3 = lax.rem(j + 2, NROWS)
            idx_wait(j + 2, r3, lax.rem(j + 2, NDIDX))
            pltpu.async_copy(*gather_refs(r3))
        return 0
    lax.fori_loop(0, nchunks, step, 0)
    pltpu.make_async_copy(
        *scat_refs(lax.rem(nchunks - 1, NROWS),
                   lax.rem(nchunks - 1, NDIDX))).wait()

    plsc.subcore_barrier()
    obase = pl.multiple_of(c * NPA + s * RPTA, 8)
    pltpu.sync_copy(agg_sh.at[pl.ds(s * RPTA, RPTA)], out_hbm.at[pl.ds(obase, RPTA)])


def _scale_of(deg_ref):
    return lax.rsqrt(jnp.maximum(deg_ref[...], 1.0))


def _f0_body(x_ref, do_ref, w_ref, out_ref):
    out_ref[0:N, :] = lax.dot_general(x_ref[...] * _scale_of(do_ref), w_ref[...],
                                      (((1,), (1,)), ((), ())),
                                      preferred_element_type=jnp.float32)


def _bn_relu(agg_ref, di_ref, g_ref, b_ref):
    a = agg_ref[...]
    h = a[0:N] + a[NPA:NPA + N]
    h = h * _scale_of(di_ref)
    m = jnp.mean(h, axis=0, keepdims=True)
    d = h - m
    v = jnp.mean(d * d, axis=0, keepdims=True)
    hn = d * lax.rsqrt(v + 1e-5) * g_ref[...] + b_ref[...]
    return jnp.maximum(hn, 0.0)


def _f1_body(agg_ref, do_ref, di_ref, g_ref, b_ref, w_ref, out_ref):
    hr = _bn_relu(agg_ref, di_ref, g_ref, b_ref)
    out_ref[0:N, :] = lax.dot_general(hr * _scale_of(do_ref), w_ref[...],
                                      (((1,), (1,)), ((), ())),
                                      preferred_element_type=jnp.float32)


def _fc_body(agg_ref, di_ref, g_ref, b_ref, wc_ref, bc_ref, out_ref):
    hr = _bn_relu(agg_ref, di_ref, g_ref, b_ref)
    out_ref[...] = lax.dot_general(hr, wc_ref[...],
                                   (((1,), (1,)), ((), ())),
                                   preferred_element_type=jnp.float32) + bc_ref[...]


_f0 = pl.pallas_call(_f0_body, out_shape=jax.ShapeDtypeStruct((NP, D), jnp.float32))
_f1 = pl.pallas_call(_f1_body, out_shape=jax.ShapeDtypeStruct((NP, D), jnp.float32))
_fc = pl.pallas_call(_fc_body, out_shape=jax.ShapeDtypeStruct((N, 40), jnp.float32))


def kernel(x, edge_index, W0, g0, b0, W1, g1, b1, W2, g2, b2, Wc, bc):
    eidx = edge_index.reshape(2 * E)
    zeros1 = jnp.zeros((NP,), jnp.float32)
    zeros128 = jnp.zeros((NPA, D), jnp.float32)
    g0r, g1r, g2r = g0.reshape(1, D), g1.reshape(1, D), g2.reshape(1, D)
    b0r, b1r, b2r = b0.reshape(1, D), b1.reshape(1, D), b2.reshape(1, D)
    bcr = bc.reshape(1, 40)

    degs = _deg_kernel(eidx, zeros1, jnp.ones((CH,), jnp.float32))
    do = degs[0:N].reshape(N, 1)
    di = degs[NP:NP + N].reshape(N, 1)

    hw = _f0(x, do, W0)
    agg = _scatter_kernel(hw, eidx, zeros128)
    hw = _f1(agg, do, di, g0r, b0r, W1)
    agg = _scatter_kernel(hw, eidx, zeros128)
    hw = _f1(agg, do, di, g1r, b1r, W2)
    agg = _scatter_kernel(hw, eidx, zeros128)
    return _fc(agg, di, g2r, b2r, Wc, bcr)
